# Initial kernel scaffold; baseline (speedup 1.0000x reference)
import jax
import jax.numpy as jnp
from jax.experimental import pallas as pl


def kernel(cut_positions, cut_local_reflatentxgene_ix, cut_local_gene_ix, cut_local_reflatent_ix, mixture_delta_reflatentxgene, unnormalized_heights, unnormalized_widths):
    raise NotImplementedError("write your pallas kernel here")


# speedup vs baseline: 1.4821x; 1.4821x over previous
import functools

import jax
import jax.numpy as jnp
from jax import lax
from jax.experimental import pallas as pl
from jax.experimental.pallas import tpu as pltpu
from jax.experimental.pallas import tpu_sc as plsc

_NBINS = (64, 32)
_NG = 5000
_NR = 16
_NRXG = _NR * _NG
_NPAD = 524288
_B = 1024
_F32 = jnp.float32
_I32 = jnp.int32


def _tables(dh_all, uh_all, uw_all):
    gs = jnp.full((_NR, _NG), 1.0 / _NG, _F32)
    out = []
    hdr = None
    h_off = 0
    w_off = 0
    for li, n in enumerate(_NBINS):
        uh = uh_all[:, h_off:h_off + n]
        uw = uw_all[:, w_off:w_off + (n - 1)]
        dh = dh_all[..., h_off:h_off + n]
        h_off += n
        w_off += n - 1
        w = jax.nn.softmax(uw, axis=-1)
        cumw = jnp.concatenate(
            [jnp.zeros((_NG, 1), _F32), jnp.cumsum(w, -1)], -1)
        eh = jnp.exp(uh[None] + dh)
        tz = (eh[..., :-1] + eh[..., 1:]) * 0.5 * w[None]
        pergene = jnp.sum(tz, -1) * gs
        area = jnp.sum(pergene, -1, keepdims=True)
        h = eh / area[..., None]
        gm = pergene / area
        C = jnp.concatenate(
            [jnp.zeros((_NR, 1), _F32), jnp.cumsum(gm, -1)], -1)[:, :-1]
        cdfl = jnp.concatenate(
            [jnp.zeros((_NR, _NG, 1), _F32),
             jnp.cumsum(tz / area[..., None], -1)], -1) * gs[..., None]
        blcdf = C[..., None] + cdfl
        out.append((cumw.reshape(-1), h.reshape(-1), blcdf.reshape(-1)))
        if li == 0:
            hdr = (C.reshape(-1), gm.reshape(-1))
        gs = gm
    return out, hdr


def _iota16():
    return lax.broadcasted_iota(_I32, (16,), 0)


def _sc_eval(xp, rxgp, gp, tabs):
    cumw0, h0, bl0, cumw1, h1, bl1, gl2t, gw2t = tabs
    info = plsc.get_sparse_core_info()
    nw = info.num_cores * info.num_subcores
    ch = _NPAD // nw
    nsub = ch // _B
    nv = _B // 16
    nck = _B // 128
    mesh = plsc.VectorSubcoreMesh(core_axis_name="c", subcore_axis_name="s")

    @functools.partial(
        pl.kernel,
        mesh=mesh,
        out_type=[jax.ShapeDtypeStruct((_NPAD,), _F32),
                  jax.ShapeDtypeStruct((_NPAD,), _F32)],
        scratch_types=[
            pltpu.VMEM((_B,), _F32),
            pltpu.VMEM((_B,), _I32),
            pltpu.VMEM((_B,), _I32),
            pltpu.VMEM((_B,), _F32),
            pltpu.VMEM((_B,), _I32),
            pltpu.VMEM((_B,), _I32),
            pltpu.VMEM((_B,), _I32),
            pltpu.VMEM((_B,), _F32),
            pltpu.VMEM((_B,), _F32),
            pltpu.VMEM((_B,), _F32),
            pltpu.VMEM((_B,), _F32),
            pltpu.VMEM((_B,), _F32),
            pltpu.VMEM((_B,), _F32),
            pltpu.VMEM((_B,), _F32),
            pltpu.VMEM((_B,), _F32),
            pltpu.SemaphoreType.DMA,
        ],
    )
    def k(x_hbm, rxg_hbm, g_hbm, cumw0_hbm, h0_hbm, bl0_hbm,
          cumw1_hbm, h1_hbm, bl1_hbm, gl2_hbm, gw2_hbm,
          out_hbm, dp_hbm,
          x_v, rxg_v, g_v, t_v, c_v, ia_v, ib_v, val_v, cwk_v, h_v, hn_v,
          bl_v, gl_v, gw_v, dp_v, sem):
        wid = lax.axis_index("s") * info.num_cores + lax.axis_index("c")

        def vloop(body):
            lax.fori_loop(0, nv, lambda i, _: (body(i, pl.ds(i * 16, 16)), 0)[1], 0)

        def rnd(tab, idx_v, dst_v):
            cps = [pltpu.async_copy(tab.at[idx_v.at[pl.ds(c * 128, 128)]],
                                    dst_v.at[pl.ds(c * 128, 128)], sem)
                   for c in range(nck)]
            for cp in cps:
                cp.wait()

        def layer(li, n, cumw_t, h_t, bl_t):
            bits = [32, 16, 8, 4, 2, 1] if n == 64 else [16, 8, 4, 2, 1]

            def init(i, sl):
                x = x_v[sl]
                if li == 0:
                    gl = g_v[sl].astype(_F32) / _F32(_NG)
                    gw = jnp.full((16,), 1.0 / _NG, _F32)
                else:
                    gl = gl_v[sl]
                    gw = gw_v[sl]
                t = (x - gl) / gw
                t_v[sl] = t
                c_v[sl] = jnp.zeros((16,), _I32)
                ia_v[sl] = g_v[sl] * n + (bits[0] - 1)

            vloop(init)
            rnd(cumw_t, ia_v, val_v)
            for bi in range(1, len(bits)):
                b_prev, b = bits[bi - 1], bits[bi]

                def step(i, sl, b_prev=b_prev, b=b):
                    c = c_v[sl] + jnp.where(
                        val_v[sl] < t_v[sl],
                        jnp.full((16,), b_prev, _I32), jnp.zeros((16,), _I32))
                    c_v[sl] = c
                    ia_v[sl] = g_v[sl] * n + (c + (b - 1))

                vloop(step)
                rnd(cumw_t, ia_v, val_v)

            def fin(i, sl):
                c = c_v[sl] + jnp.where(
                    val_v[sl] < t_v[sl],
                    jnp.ones((16,), _I32), jnp.zeros((16,), _I32))
                kk = jnp.clip(c - 1, 0, n - 2)
                c_v[sl] = kk
                ia_v[sl] = g_v[sl] * n + kk
                ib_v[sl] = rxg_v[sl] * n + kk

            vloop(fin)
            rnd(cumw_t, ia_v, cwk_v)
            rnd(h_t, ib_v, h_v)
            rnd(bl_t, ib_v, bl_v)

            def bump(i, sl):
                ia_v[sl] = ia_v[sl] + jnp.ones((16,), _I32)
                ib_v[sl] = ib_v[sl] + jnp.ones((16,), _I32)

            vloop(bump)
            rnd(cumw_t, ia_v, val_v)
            rnd(h_t, ib_v, hn_v)

            def apply(i, sl):
                x = x_v[sl]
                if li == 0:
                    gl = g_v[sl].astype(_F32) / _F32(_NG)
                    gw = jnp.full((16,), 1.0 / _NG, _F32)
                else:
                    gl = gl_v[sl]
                    gw = gw_v[sl]
                cwk = cwk_v[sl]
                bw = gw * (val_v[sl] - cwk)
                left = gl + gw * cwk
                a = jnp.clip((x - left) / jnp.maximum(bw, _F32(1e-12)),
                             0.0, 1.0)
                hl = h_v[sl]
                dh = hn_v[sl] - hl
                x_v[sl] = bl_v[sl] + a * bw * hl + _F32(0.5) * a * a * bw * dh
                f = jnp.maximum(hl + a * dh, _F32(1e-12))
                if li == 0:
                    dp_v[sl] = f
                else:
                    dp_v[sl] = dp_v[sl] * f

            vloop(apply)

        def sub(s, _):
            base = wid * ch + s * _B
            pltpu.sync_copy(x_hbm.at[pl.ds(base, _B)], x_v)
            pltpu.sync_copy(rxg_hbm.at[pl.ds(base, _B)], rxg_v)
            pltpu.sync_copy(g_hbm.at[pl.ds(base, _B)], g_v)
            layer(0, 64, cumw0_hbm, h0_hbm, bl0_hbm)
            rnd(gl2_hbm, rxg_v, gl_v)
            rnd(gw2_hbm, rxg_v, gw_v)
            layer(1, 32, cumw1_hbm, h1_hbm, bl1_hbm)
            pltpu.sync_copy(x_v, out_hbm.at[pl.ds(base, _B)])
            pltpu.sync_copy(dp_v, dp_hbm.at[pl.ds(base, _B)])
            return 0

        lax.fori_loop(0, nsub, sub, 0)

    return k(xp, rxgp, gp, cumw0, h0, bl0, cumw1, h1, bl1, gl2t, gw2t)


def _log_kernel(dp_ref, o_ref):
    o_ref[...] = jnp.log(dp_ref[...])


def kernel(cut_positions, cut_local_reflatentxgene_ix, cut_local_gene_ix,
           cut_local_reflatent_ix, mixture_delta_reflatentxgene,
           unnormalized_heights, unnormalized_widths):
    del cut_local_reflatent_ix
    n = cut_positions.shape[0]
    (l0, l1), (gl2t, gw2t) = _tables(mixture_delta_reflatentxgene,
                                     unnormalized_heights, unnormalized_widths)
    pad = _NPAD - n
    xp = jnp.pad(cut_positions, (0, pad))
    rxgp = jnp.pad(cut_local_reflatentxgene_ix.astype(_I32), (0, pad))
    gp = jnp.pad(cut_local_gene_ix.astype(_I32), (0, pad))
    out_p, dp_p = _sc_eval(xp, rxgp, gp, (*l0, *l1, gl2t, gw2t))
    lad_p = pl.pallas_call(
        _log_kernel,
        out_shape=jax.ShapeDtypeStruct((_NPAD // 128, 128), _F32),
    )(dp_p.reshape(_NPAD // 128, 128)).reshape(-1)
    return out_p[:n], lad_p[:n]
